# Initial kernel scaffold; baseline (speedup 1.0000x reference)
import jax
import jax.numpy as jnp
from jax.experimental import pallas as pl


def kernel(features, codebook):
    raise NotImplementedError("write your pallas kernel here")


# speedup vs baseline: 1.3253x; 1.3253x over previous
import functools

import jax
import jax.numpy as jnp
from jax import lax
from jax.experimental import pallas as pl
from jax.experimental.pallas import tpu as pltpu
from jax.experimental.pallas import tpu_sc as plsc

B = 32768
D = 32
C = 8192

M_TILE = 512
CHUNK = 2048
N_CHUNKS = C // CHUNK

NW = 32
BPW = B // NW
GW = 128
NG = BPW // GW


def _argmin_body(f_ref, cbt_ref, idx_ref):
    f = f_ref[...]
    a = jnp.sum(f * f, axis=1, keepdims=True)
    bestv = jnp.full((M_TILE, 1), jnp.inf, jnp.float32)
    besti = jnp.zeros((M_TILE, 1), jnp.int32)
    for j in range(N_CHUNKS):
        ch = cbt_ref[:, j * CHUNK:(j + 1) * CHUNK]
        b = jnp.sum(ch * ch, axis=0, keepdims=True)
        ch_r = ch.astype(jnp.bfloat16).astype(jnp.float32)
        t = lax.dot_general(
            f, ch_r, (((1,), (0,)), ((), ())),
            preferred_element_type=jnp.float32,
        )
        dist = (a + b) - 2.0 * t
        m = jnp.min(dist, axis=1, keepdims=True)
        io = lax.broadcasted_iota(jnp.int32, dist.shape, 1)
        cand = jnp.where(dist == m, io, jnp.int32(2 ** 30))
        li = jnp.min(cand, axis=1, keepdims=True) + j * CHUNK
        upd = (m < bestv) | ((m == bestv) & (li < besti))
        bestv = jnp.where(upd, m, bestv)
        besti = jnp.where(upd, li, besti)
    idx_ref[...] = besti[:, 0]


def _argmin_call(features, cbt):
    return pl.pallas_call(
        _argmin_body,
        grid=(B // M_TILE,),
        in_specs=[
            pl.BlockSpec((M_TILE, D), lambda i: (i, 0)),
            pl.BlockSpec((D, C), lambda i: (0, 0)),
        ],
        out_specs=pl.BlockSpec((M_TILE,), lambda i: (i,)),
        out_shape=jax.ShapeDtypeStruct((B,), jnp.int32),
        compiler_params=pltpu.CompilerParams(
            dimension_semantics=("parallel",)),
    )(features, cbt)


F_TILE = 4096


def _final_body(f_ref, q_ref, cnt_ref, fr_ref, loss_ref, ent_ref, acc_ref):
    i = pl.program_id(0)
    f = f_ref[...]
    q = q_ref[...]
    d = q - f
    fr_ref[...] = f + d
    part = jnp.sum(d * d)

    @pl.when(i == 0)
    def _():
        acc_ref[0] = part

    @pl.when(i > 0)
    def _():
        acc_ref[0] = acc_ref[0] + part

    @pl.when(i == pl.num_programs(0) - 1)
    def _():
        loss = acc_ref[0] / (B * D)
        loss_ref[...] = jnp.full((8, 128), loss, jnp.float32)
        p = cnt_ref[...] / B
        ent = -jnp.sum(p * jnp.log(p + 1e-08))
        ent_ref[...] = jnp.full((8, 128), ent, jnp.float32)


def _final_call(features, q, cnt2d):
    return pl.pallas_call(
        _final_body,
        grid=(B // F_TILE,),
        in_specs=[
            pl.BlockSpec((F_TILE, D), lambda i: (i, 0)),
            pl.BlockSpec((F_TILE, D), lambda i: (i, 0)),
            pl.BlockSpec((C // 128, 128), lambda i: (0, 0)),
        ],
        out_specs=[
            pl.BlockSpec((F_TILE, D), lambda i: (i, 0)),
            pl.BlockSpec((8, 128), lambda i: (0, 0)),
            pl.BlockSpec((8, 128), lambda i: (0, 0)),
        ],
        out_shape=[
            jax.ShapeDtypeStruct((B, D), jnp.float32),
            jax.ShapeDtypeStruct((8, 128), jnp.float32),
            jax.ShapeDtypeStruct((8, 128), jnp.float32),
        ],
        scratch_shapes=[pltpu.SMEM((1,), jnp.float32)],
        compiler_params=pltpu.CompilerParams(
            dimension_semantics=("arbitrary",)),
    )(features, q, cnt2d)


def _sc_gather_hist(codebook, idx3, ones_blk, zeros_init):
    mesh = plsc.VectorSubcoreMesh(core_axis_name="c", subcore_axis_name="s")

    @functools.partial(
        pl.kernel,
        out_type=[
            jax.ShapeDtypeStruct((B, D), jnp.float32),
            jax.ShapeDtypeStruct((2, C, 16), jnp.float32),
        ],
        mesh=mesh,
        scratch_types=[
            pltpu.VMEM((NG, GW), jnp.int32),
            pltpu.VMEM((BPW, D), jnp.float32),
            pltpu.VMEM((GW, 16), jnp.float32),
            pltpu.VMEM_SHARED((C, 16), jnp.float32),
            pltpu.SemaphoreType.DMA,
        ],
        compiler_params=pltpu.CompilerParams(use_tc_tiling_on_sc=False),
    )
    def k(cb_hbm, idx_hbm, ones_hbm, z_hbm, q_hbm, cnt_hbm,
          idx_v, rows_v, ones_v, shared, sem):
        cid = lax.axis_index("c")
        sid = lax.axis_index("s")
        wid = cid * 16 + sid
        pltpu.sync_copy(idx_hbm.at[wid], idx_v)
        pltpu.sync_copy(ones_hbm, ones_v)
        stripe = C // 16
        pltpu.sync_copy(z_hbm.at[pl.ds(sid * stripe, stripe)],
                        shared.at[pl.ds(sid * stripe, stripe)])
        plsc.subcore_barrier()
        copies = []
        for j in range(NG):
            copies.append(pltpu.async_copy(
                cb_hbm.at[idx_v.at[j]],
                rows_v.at[pl.ds(j * GW, GW)], sem))
        for cpy in copies:
            cpy.wait()
        pltpu.sync_copy(rows_v, q_hbm.at[pl.ds(wid * BPW, BPW)])
        for j in range(NG):
            pltpu.sync_copy(ones_v, shared.at[idx_v.at[j]], add=True)
        plsc.subcore_barrier()

        @pl.when(sid == 0)
        def _():
            pltpu.sync_copy(shared, cnt_hbm.at[cid])

    return k(codebook, idx3, ones_blk, zeros_init)


def kernel(features, codebook):
    cbt = codebook.T
    idx = _argmin_call(features, cbt)
    idx3 = idx.reshape(NW, NG, GW)
    ones_blk = jnp.zeros((GW, 16), jnp.float32).at[:, 0].set(1.0)
    zeros_init = jnp.zeros((C, 16), jnp.float32)
    q, counts16 = _sc_gather_hist(codebook, idx3, ones_blk, zeros_init)
    cnt2d = (counts16[0, :, 0] + counts16[1, :, 0]).reshape(C // 128, 128)
    fr, loss, ent = _final_call(features, q, cnt2d)
    loss = loss[0, 0]
    return (fr, loss, loss, ent[0, 0])


# speedup vs baseline: 1.3666x; 1.0311x over previous
import functools

import jax
import jax.numpy as jnp
from jax import lax
from jax.experimental import pallas as pl
from jax.experimental.pallas import tpu as pltpu
from jax.experimental.pallas import tpu_sc as plsc

B = 32768
D = 32
C = 8192

M_TILE = 1024
CHUNK = 2048
N_CHUNKS = C // CHUNK

NW = 32
BPW = B // NW
GW = 128
NG = BPW // GW


def _argmin_body(f_ref, cbt_ref, idx_ref):
    f = f_ref[...]
    a = jnp.sum(f * f, axis=1, keepdims=True)
    bestv = jnp.full((M_TILE, 1), jnp.inf, jnp.float32)
    besti = jnp.zeros((M_TILE, 1), jnp.int32)
    for j in range(N_CHUNKS):
        ch = cbt_ref[:, j * CHUNK:(j + 1) * CHUNK]
        b = jnp.sum(ch * ch, axis=0, keepdims=True)
        ch_r = ch.astype(jnp.bfloat16).astype(jnp.float32)
        t = lax.dot_general(
            f, ch_r, (((1,), (0,)), ((), ())),
            preferred_element_type=jnp.float32,
        )
        dist = (a + b) - 2.0 * t
        m = jnp.min(dist, axis=1, keepdims=True)
        io = lax.broadcasted_iota(jnp.int32, dist.shape, 1)
        cand = jnp.where(dist == m, io, jnp.int32(2 ** 30))
        li = jnp.min(cand, axis=1, keepdims=True) + j * CHUNK
        upd = (m < bestv) | ((m == bestv) & (li < besti))
        bestv = jnp.where(upd, m, bestv)
        besti = jnp.where(upd, li, besti)
    idx_ref[...] = besti[:, 0]


def _argmin_call(features, cbt):
    return pl.pallas_call(
        _argmin_body,
        grid=(B // M_TILE,),
        in_specs=[
            pl.BlockSpec((M_TILE, D), lambda i: (i, 0)),
            pl.BlockSpec((D, C), lambda i: (0, 0)),
        ],
        out_specs=pl.BlockSpec((M_TILE,), lambda i: (i,)),
        out_shape=jax.ShapeDtypeStruct((B,), jnp.int32),
        compiler_params=pltpu.CompilerParams(
            dimension_semantics=("parallel",)),
    )(features, cbt)


F_TILE = 4096


def _final_body(f_ref, q_ref, cnt_ref, fr_ref, loss_ref, ent_ref, acc_ref):
    i = pl.program_id(0)
    f = f_ref[...]
    q = q_ref[...]
    d = q - f
    fr_ref[...] = f + d
    part = jnp.sum(d * d)

    @pl.when(i == 0)
    def _():
        acc_ref[0] = part

    @pl.when(i > 0)
    def _():
        acc_ref[0] = acc_ref[0] + part

    @pl.when(i == pl.num_programs(0) - 1)
    def _():
        loss = acc_ref[0] / (B * D)
        loss_ref[...] = jnp.full((8, 128), loss, jnp.float32)
        p = cnt_ref[...] / B
        ent = -jnp.sum(p * jnp.log(p + 1e-08))
        ent_ref[...] = jnp.full((8, 128), ent, jnp.float32)


def _final_call(features, q, cnt2d):
    return pl.pallas_call(
        _final_body,
        grid=(B // F_TILE,),
        in_specs=[
            pl.BlockSpec((F_TILE, D), lambda i: (i, 0)),
            pl.BlockSpec((F_TILE, D), lambda i: (i, 0)),
            pl.BlockSpec((C // 128, 128), lambda i: (0, 0)),
        ],
        out_specs=[
            pl.BlockSpec((F_TILE, D), lambda i: (i, 0)),
            pl.BlockSpec((8, 128), lambda i: (0, 0)),
            pl.BlockSpec((8, 128), lambda i: (0, 0)),
        ],
        out_shape=[
            jax.ShapeDtypeStruct((B, D), jnp.float32),
            jax.ShapeDtypeStruct((8, 128), jnp.float32),
            jax.ShapeDtypeStruct((8, 128), jnp.float32),
        ],
        scratch_shapes=[pltpu.SMEM((1,), jnp.float32)],
        compiler_params=pltpu.CompilerParams(
            dimension_semantics=("arbitrary",)),
    )(features, q, cnt2d)


def _sc_gather_hist(codebook, idx3, ones_blk, zeros_init):
    mesh = plsc.VectorSubcoreMesh(core_axis_name="c", subcore_axis_name="s")

    @functools.partial(
        pl.kernel,
        out_type=[
            jax.ShapeDtypeStruct((B, D), jnp.float32),
            jax.ShapeDtypeStruct((2, C, 16), jnp.float32),
        ],
        mesh=mesh,
        scratch_types=[
            pltpu.VMEM((NG, GW), jnp.int32),
            pltpu.VMEM((BPW, D), jnp.float32),
            pltpu.VMEM((GW, 16), jnp.float32),
            pltpu.VMEM_SHARED((C, 16), jnp.float32),
            pltpu.SemaphoreType.DMA,
        ],
        compiler_params=pltpu.CompilerParams(use_tc_tiling_on_sc=False),
    )
    def k(cb_hbm, idx_hbm, ones_hbm, z_hbm, q_hbm, cnt_hbm,
          idx_v, rows_v, ones_v, shared, sem):
        cid = lax.axis_index("c")
        sid = lax.axis_index("s")
        wid = cid * 16 + sid
        pltpu.sync_copy(idx_hbm.at[wid], idx_v)
        pltpu.sync_copy(ones_hbm, ones_v)
        stripe = C // 16
        pltpu.sync_copy(z_hbm.at[pl.ds(sid * stripe, stripe)],
                        shared.at[pl.ds(sid * stripe, stripe)])
        plsc.subcore_barrier()
        copies = []
        for j in range(NG):
            copies.append(pltpu.async_copy(
                cb_hbm.at[idx_v.at[j]],
                rows_v.at[pl.ds(j * GW, GW)], sem))
        for cpy in copies:
            cpy.wait()
        pltpu.sync_copy(rows_v, q_hbm.at[pl.ds(wid * BPW, BPW)])
        for j in range(NG):
            pltpu.sync_copy(ones_v, shared.at[idx_v.at[j]], add=True)
        plsc.subcore_barrier()

        @pl.when(sid == 0)
        def _():
            pltpu.sync_copy(shared, cnt_hbm.at[cid])

    return k(codebook, idx3, ones_blk, zeros_init)


def kernel(features, codebook):
    cbt = codebook.T
    idx = _argmin_call(features, cbt)
    idx3 = idx.reshape(NW, NG, GW)
    ones_blk = jnp.zeros((GW, 16), jnp.float32).at[:, 0].set(1.0)
    zeros_init = jnp.zeros((C, 16), jnp.float32)
    q, counts16 = _sc_gather_hist(codebook, idx3, ones_blk, zeros_init)
    cnt2d = (counts16[0, :, 0] + counts16[1, :, 0]).reshape(C // 128, 128)
    fr, loss, ent = _final_call(features, q, cnt2d)
    loss = loss[0, 0]
    return (fr, loss, loss, ent[0, 0])
